# Initial kernel scaffold; baseline (speedup 1.0000x reference)
import jax
import jax.numpy as jnp
from jax.experimental import pallas as pl


def kernel(x, W_ih, W_hh, b_ih, b_hh, fo_W, fo_b, rf_W, rf_b, g_W1, g_b1, g_W2, g_b2, meta_sequences, meta_labels):
    raise NotImplementedError("write your pallas kernel here")


# speedup vs baseline: 1.7983x; 1.7983x over previous
import functools

import jax
import jax.numpy as jnp
from jax import lax
from jax.experimental import pallas as pl
from jax.experimental.pallas import tpu as pltpu

B, T, F, H, FS, OD, TK = 1024, 50, 8, 64, 8, 1, 4
M = 100000
MT = 2048
M_PAD = ((M + MT - 1) // MT) * MT
N_TILES = M_PAD // MT
BIG_I = 2**30
INF = float("inf")


def _lex_lt(v1, i1, v2, i2):
    return (v1 < v2) | ((v1 == v2) & (i1 < i2))


def _topk_kernel(q_ref, metaT_ref, idx_ref, bv_ref, bi_ref):
    pid = pl.program_id(0)

    @pl.when(pid == 0)
    def _init():
        bv_ref[...] = jnp.full((B, 8), INF, jnp.float32)
        bi_ref[...] = jnp.full((B, 8), BIG_I, jnp.int32)

    mt = metaT_ref[...]
    norms = jnp.sum(mt * mt, axis=0, keepdims=True)
    qm = jnp.dot(q_ref[...], mt, preferred_element_type=jnp.float32)
    s = norms - 2.0 * qm

    lane = lax.broadcasted_iota(jnp.int32, (1, MT), 1)
    off = pid * MT
    gidx = lane + off
    s = jnp.where(gidx < M, s, INF)

    new_v, new_i = [], []
    for _ in range(TK):
        m = jnp.min(s, axis=1, keepdims=True)
        am = jnp.min(jnp.where(s == m, lane, BIG_I), axis=1, keepdims=True)
        new_v.append(m)
        new_i.append(am + off)
        s = jnp.where(lane == am, INF, s)

    cand_v = [bv_ref[:, k][:, None] for k in range(TK)] + new_v
    cand_i = [bi_ref[:, k][:, None] for k in range(TK)] + new_i
    out_v, out_i = [], []
    for _ in range(TK):
        best_v, best_i = cand_v[0], cand_i[0]
        for j in range(1, 2 * TK):
            take = _lex_lt(cand_v[j], cand_i[j], best_v, best_i)
            best_v = jnp.where(take, cand_v[j], best_v)
            best_i = jnp.where(take, cand_i[j], best_i)
        out_v.append(best_v)
        out_i.append(best_i)
        for j in range(2 * TK):
            hit = cand_i[j] == best_i
            cand_v[j] = jnp.where(hit, INF, cand_v[j])
            cand_i[j] = jnp.where(hit, BIG_I, cand_i[j])
    pad_v = [jnp.full((B, 1), INF, jnp.float32)] * (8 - TK)
    pad_i = [jnp.full((B, 1), BIG_I, jnp.int32)] * (8 - TK)
    bv_ref[...] = jnp.concatenate(out_v + pad_v, axis=1)
    bi_ref[...] = jnp.concatenate(out_i + pad_i, axis=1)

    @pl.when(pid == N_TILES - 1)
    def _fin():
        idx_ref[...] = bi_ref[...]


def _topk(q, metaT_pad):
    return pl.pallas_call(
        _topk_kernel,
        grid=(N_TILES,),
        in_specs=[
            pl.BlockSpec((B, T), lambda i: (0, 0)),
            pl.BlockSpec((T, MT), lambda i: (0, i)),
        ],
        out_specs=pl.BlockSpec((B, 8), lambda i: (0, 0)),
        out_shape=jax.ShapeDtypeStruct((B, 8), jnp.int32),
        scratch_shapes=[
            pltpu.VMEM((B, 8), jnp.float32),
            pltpu.VMEM((B, 8), jnp.int32),
        ],
    )(q, metaT_pad)


def _forecast_kernel(xT_ref, q_ref, rag_ref,
                     WihT_ref, WhhT_ref, bih_ref, bhh_ref,
                     foWT_ref, fob_ref, rfWT_ref, rfb_ref,
                     gW1T_ref, gb1_ref, gW2T_ref, gb2_ref,
                     out_ref):
    WihT = WihT_ref[...]
    WhhT = WhhT_ref[...]
    bih = bih_ref[...]
    bhh = bhh_ref[...]

    def step(t, h):
        xt = xT_ref[t]
        gi = jnp.dot(xt, WihT, preferred_element_type=jnp.float32) + bih
        gh = jnp.dot(h, WhhT, preferred_element_type=jnp.float32) + bhh
        r = jax.nn.sigmoid(gi[:, :H] + gh[:, :H])
        z = jax.nn.sigmoid(gi[:, H:2 * H] + gh[:, H:2 * H])
        n = jnp.tanh(gi[:, 2 * H:] + r * gh[:, 2 * H:])
        return (1.0 - z) * n + z * h

    h = lax.fori_loop(0, T, step, jnp.zeros((B, H), jnp.float32))

    q = q_ref[...]
    t1 = jnp.tanh(jnp.dot(q, gW1T_ref[...], preferred_element_type=jnp.float32) + gb1_ref[...])
    gate = jax.nn.sigmoid(jnp.dot(t1, gW2T_ref[...], preferred_element_type=jnp.float32) + gb2_ref[...])
    ragl = jnp.dot(rag_ref[...], rfWT_ref[...], preferred_element_type=jnp.float32) + rfb_ref[...]
    out = jnp.dot(h, foWT_ref[...], preferred_element_type=jnp.float32) + fob_ref[...]
    out_ref[...] = out + gate * ragl


def _forecast(xT, q, rag_flat, WihT, WhhT, bih, bhh, foWT, fob, rfWT, rfb,
              gW1T, gb1, gW2T, gb2):
    return pl.pallas_call(
        _forecast_kernel,
        out_shape=jax.ShapeDtypeStruct((B, FS * OD), jnp.float32),
    )(xT, q, rag_flat, WihT, WhhT, bih, bhh, foWT, fob, rfWT, rfb,
      gW1T, gb1, gW2T, gb2)


@jax.jit
def kernel(x, W_ih, W_hh, b_ih, b_hh, fo_W, fo_b, rf_W, rf_b,
           g_W1, g_b1, g_W2, g_b2, meta_sequences, meta_labels):
    q = x[:, :, 3]
    metaT = jnp.pad(meta_sequences.T, ((0, 0), (0, M_PAD - M)))
    idx8 = _topk(q, metaT)
    idx = idx8[:, :TK]

    rag_refs = jnp.take(meta_labels, idx, axis=0)
    rag_flat = rag_refs.reshape(B, TK * FS)

    xT = jnp.transpose(x, (1, 0, 2))
    out = _forecast(
        xT, q, rag_flat,
        W_ih.T, W_hh.T, b_ih[None, :], b_hh[None, :],
        fo_W.T, fo_b[None, :], rf_W.T, rf_b[None, :],
        g_W1.T, g_b1[None, :], g_W2.T, g_b2[None, :],
    )
    return out
